# Initial kernel scaffold; baseline (speedup 1.0000x reference)
import jax
import jax.numpy as jnp
from jax.experimental import pallas as pl


def kernel(x, onehot0, edge_index, batch_sample_indices, n_sample_nodes, adj0, W, att_l, att_r, bias):
    raise NotImplementedError("write your pallas kernel here")


# speedup vs baseline: 19.0893x; 19.0893x over previous
import functools

import jax
import jax.numpy as jnp
from jax import lax
from jax.experimental import pallas as pl
from jax.experimental.pallas import tpu as pltpu
from jax.experimental.pallas import tpu_sc as plsc

_NEG_SLOPE = 0.2
_CHUNK = 128


def _ceil_to(a, m):
    return (a + m - 1) // m * m


def _prescale_body(x_ref, wt_ref, att_ref, oh_ref, ya_ref, yb_ref):
    xb = jnp.dot(x_ref[...], wt_ref[...], preferred_element_type=jnp.float32)
    s = jnp.sum(xb * att_ref[...], axis=1, keepdims=True)
    g = jnp.where(s >= 0, s, _NEG_SLOPE * s)
    eg = jnp.exp(g)
    y = xb * eg
    egoh = oh_ref[...] * eg
    ya_ref[...] = jnp.concatenate([y[:, :96], egoh[:, :32]], axis=1)
    yb_ref[...] = jnp.concatenate(
        [y[:, 96:], egoh[:, 32:], jnp.zeros_like(y[:, :64])], axis=1)


def _finalize_body(a_ref, b_ref, bias_ref, outx_ref, outoh_ref):
    a = a_ref[...]
    b = b_ref[...]
    oh = jnp.concatenate([a[:, 96:], b[:, 32:64]], axis=1)
    den = jnp.sum(oh, axis=1, keepdims=True)
    inv = 1.0 / (den + 1e-16)
    outx_ref[...] = jnp.concatenate([a[:, :96], b[:, :32]], axis=1) * inv + bias_ref[...]
    outoh_ref[...] = oh * inv


def _make_sc_scatter(n_nodes, n_pad, e_pad, feat):
    n_tiles = 16
    ept = e_pad // n_tiles
    assert ept * n_tiles == e_pad and ept % _CHUNK == 0
    n_chunks = ept // _CHUNK
    rows_pt = n_pad // n_tiles

    mesh = plsc.VectorSubcoreMesh(core_axis_name="c", subcore_axis_name="s")

    @functools.partial(
        pl.kernel,
        mesh=mesh,
        out_type=jax.ShapeDtypeStruct((2 * n_pad, feat), jnp.float32),
        scratch_types=[
            pltpu.VMEM((_CHUNK,), jnp.int32),
            pltpu.VMEM((_CHUNK,), jnp.int32),
            pltpu.VMEM((_CHUNK, feat), jnp.float32),
            pltpu.VMEM_SHARED((n_pad, feat), jnp.float32),
            pltpu.SemaphoreType.DMA,
        ],
    )
    def sc_scatter(ya, yb, send, recv, out, sidx, ridx, rows, acc, sem):
        cid = lax.axis_index("c")
        sid = lax.axis_index("s")

        def zero_row(i, _):
            for j in range(feat // 16):
                rows[i, pl.ds(j * 16, 16)] = jnp.zeros((16,), jnp.float32)
            return _

        lax.fori_loop(0, _CHUNK, zero_row, 0)
        rbase = sid * rows_pt
        off = 0
        while off < rows_pt:
            step = min(_CHUNK, rows_pt - off)
            pltpu.sync_copy(rows.at[pl.ds(0, step)], acc.at[pl.ds(rbase + off, step)])
            off += step
        plsc.subcore_barrier()

        ebase = sid * ept

        def chunk(c, _):
            b = ebase + c * _CHUNK
            pltpu.sync_copy(send.at[pl.ds(b, _CHUNK)], sidx)
            pltpu.sync_copy(recv.at[pl.ds(b, _CHUNK)], ridx)

            @pl.when(cid == 0)
            def _g0():
                pltpu.async_copy(ya.at[sidx], rows, sem).wait()

            @pl.when(cid == 1)
            def _g1():
                pltpu.async_copy(yb.at[sidx], rows, sem).wait()

            pltpu.sync_copy(rows, acc.at[ridx], add=True)
            return _

        lax.fori_loop(0, n_chunks, chunk, 0)
        plsc.subcore_barrier()

        obase = cid * n_pad + rbase
        off = 0
        while off < rows_pt:
            step = min(_CHUNK, rows_pt - off)
            pltpu.sync_copy(acc.at[pl.ds(rbase + off, step)], rows.at[pl.ds(0, step)])
            pltpu.sync_copy(rows.at[pl.ds(0, step)], out.at[pl.ds(obase + off, step)])
            off += step

    return sc_scatter


def kernel(x, onehot0, edge_index, batch_sample_indices, n_sample_nodes, adj0,
           W, att_l, att_r, bias):
    n, d = x.shape
    ncls = onehot0.shape[1]
    c = W.shape[0]
    e = edge_index.shape[1]
    feat = 128

    att = (att_l + att_r).reshape(1, c).astype(jnp.float32)

    blk = 400
    nb = _ceil_to(n, blk) // blk
    n_rows = nb * blk
    xr = x if n_rows == n else jnp.pad(x, ((0, n_rows - n), (0, 0)))
    ohr = onehot0 if n_rows == n else jnp.pad(onehot0, ((0, n_rows - n), (0, 0)))
    ya, yb = pl.pallas_call(
        _prescale_body,
        grid=(nb,),
        in_specs=[
            pl.BlockSpec((blk, d), lambda i: (i, 0)),
            pl.BlockSpec((d, c), lambda i: (0, 0)),
            pl.BlockSpec((1, c), lambda i: (0, 0)),
            pl.BlockSpec((blk, ncls), lambda i: (i, 0)),
        ],
        out_specs=[
            pl.BlockSpec((blk, feat), lambda i: (i, 0)),
            pl.BlockSpec((blk, feat), lambda i: (i, 0)),
        ],
        out_shape=[jax.ShapeDtypeStruct((n_rows, feat), jnp.float32)] * 2,
    )(xr, W.T, att, ohr)

    n_pad = _ceil_to(n + 1, 16 * 8)
    e_pad = _ceil_to(e, 16 * _CHUNK)
    send = edge_index[0]
    recv = edge_index[1]
    if e_pad != e:
        pad = e_pad - e
        send = jnp.concatenate([send, jnp.zeros((pad,), jnp.int32)])
        recv = jnp.concatenate([recv, jnp.full((pad,), n, jnp.int32)])
    agg = _make_sc_scatter(n, n_pad, e_pad, feat)(ya, yb, send, recv)

    agg_a = agg[:n_rows]
    agg_b = lax.dynamic_slice_in_dim(agg, n_pad, n_rows)
    out_x, agg_onehot = pl.pallas_call(
        _finalize_body,
        grid=(nb,),
        in_specs=[
            pl.BlockSpec((blk, feat), lambda i: (i, 0)),
            pl.BlockSpec((blk, feat), lambda i: (i, 0)),
            pl.BlockSpec((1, c), lambda i: (0, 0)),
        ],
        out_specs=[
            pl.BlockSpec((blk, c), lambda i: (i, 0)),
            pl.BlockSpec((blk, ncls), lambda i: (i, 0)),
        ],
        out_shape=[
            jax.ShapeDtypeStruct((n_rows, c), jnp.float32),
            jax.ShapeDtypeStruct((n_rows, ncls), jnp.float32),
        ],
    )(agg_a, agg_b, bias.reshape(1, c))
    return out_x[:n], agg_onehot[:n]
